# Initial kernel scaffold; baseline (speedup 1.0000x reference)
import jax
import jax.numpy as jnp
from jax.experimental import pallas as pl


def kernel(features, mol_index, n_molecules):
    raise NotImplementedError("write your pallas kernel here")


# speedup vs baseline: 4.2123x; 4.2123x over previous
import functools

import jax
import jax.numpy as jnp
from jax import lax
from jax.experimental import pallas as pl
from jax.experimental.pallas import tpu as pltpu
from jax.experimental.pallas import tpu_sc as plsc

N_ATOMS = 320000
D_FEAT = 128
N_MOLS = 10000

N_CORES = 2
N_SUB = 16
NW = N_CORES * N_SUB
PER_W = N_ATOMS // NW
CHUNK = 80
STEPS = PER_W // CHUNK
M_PER_SUB = N_MOLS // N_SUB
ZROWS = 125
ZCHUNKS = N_MOLS // ZROWS


def _sc_partials(features4, idx3, zeros_stage):
    mesh = plsc.VectorSubcoreMesh(core_axis_name="c", subcore_axis_name="s")

    @functools.partial(
        pl.kernel,
        out_type=jax.ShapeDtypeStruct((N_CORES, ZCHUNKS, ZROWS, D_FEAT),
                                      jnp.float32),
        mesh=mesh,
        scratch_types=[
            pltpu.VMEM((STEPS, CHUNK), jnp.int32),
            pltpu.VMEM((CHUNK, D_FEAT), jnp.float32),
            pltpu.VMEM((ZROWS, D_FEAT), jnp.float32),
            pltpu.VMEM_SHARED((N_MOLS, D_FEAT), jnp.float32),
        ],
    )
    def k(feat_hbm, idx_hbm, zero_hbm, part_hbm, idx_v, rows_v, stage_v, accum_sh):
        c = lax.axis_index("c")
        s = lax.axis_index("s")
        wid = c * N_SUB + s

        pltpu.sync_copy(zero_hbm, stage_v)
        for kk in range(M_PER_SUB // ZROWS):
            pltpu.sync_copy(stage_v,
                            accum_sh.at[pl.ds(s * M_PER_SUB + kk * ZROWS, ZROWS)])
        plsc.subcore_barrier()

        pltpu.sync_copy(idx_hbm.at[wid], idx_v)

        def step(j, carry):
            pltpu.sync_copy(feat_hbm.at[wid, j], rows_v)
            pltpu.sync_copy(rows_v, accum_sh.at[idx_v.at[j]], add=True)
            return carry

        lax.fori_loop(0, STEPS, step, 0)
        plsc.subcore_barrier()

        for kk in range(M_PER_SUB // ZROWS):
            q = s * (M_PER_SUB // ZROWS) + kk
            pltpu.sync_copy(accum_sh.at[pl.ds(q * ZROWS, ZROWS)], stage_v)
            pltpu.sync_copy(stage_v, part_hbm.at[c, q])

    return k(features4, idx3, zeros_stage)


def _combine_body(a_ref, b_ref, o_ref):
    o_ref[...] = a_ref[...] + b_ref[...]


_COMBINE_BLK = 1000


def _combine(p0, p1):
    grid = N_MOLS // _COMBINE_BLK
    spec = pl.BlockSpec((_COMBINE_BLK, D_FEAT), lambda i: (i, 0))
    return pl.pallas_call(
        _combine_body,
        grid=(grid,),
        in_specs=[spec, spec],
        out_specs=spec,
        out_shape=jax.ShapeDtypeStruct((N_MOLS, D_FEAT), jnp.float32),
    )(p0, p1)


def kernel(features, mol_index, n_molecules):
    del n_molecules
    feat4 = features.reshape(NW, STEPS, CHUNK, D_FEAT)
    idx3 = mol_index.astype(jnp.int32).reshape(NW, STEPS, CHUNK)
    zeros_stage = jnp.zeros((ZROWS, D_FEAT), jnp.float32)
    part = _sc_partials(feat4, idx3, zeros_stage)
    part = part.reshape(N_CORES, N_MOLS, D_FEAT)
    return _combine(part[0], part[1])
